# baseline (speedup 1.0000x reference)
import functools

import jax
import jax.numpy as jnp
from jax import lax
from jax.experimental import pallas as pl
from jax.experimental.pallas import tpu as pltpu
from jax.experimental.pallas import tpu_sc as plsc


def _sc_mask_body(edges_hbm, mask_hbm, e_v, m_v):
    A, DEG, B = edges_hbm.shape
    ach, bch = A // 4, 128
    w = lax.axis_index("s") * 2 + lax.axis_index("c")
    a0 = (w // 8) * ach
    b0 = (w % 8) * bch
    pltpu.sync_copy(edges_hbm.at[pl.ds(a0, ach), :, pl.ds(b0, bch)], e_v)

    def atom_body(a, carry):
        for h in range(bch // 16):
            acc = jnp.zeros((16,), jnp.int32)
            for d in range(DEG):
                e = e_v[a, d, pl.ds(h * 16, 16)]
                acc = acc | jnp.where(e != -1, jnp.int32(1), jnp.int32(0))
            m_v[a, pl.ds(h * 16, 16)] = acc.astype(jnp.float32)
        return carry

    lax.fori_loop(0, ach, atom_body, 0)
    pltpu.sync_copy(m_v, mask_hbm.at[pl.ds(a0, ach), pl.ds(b0, bch)])


def _sc_mask(edges_t):
    A, DEG, B = edges_t.shape
    ach, bch = A // 4, 128
    mesh = plsc.VectorSubcoreMesh(core_axis_name="c", subcore_axis_name="s")
    return functools.partial(
        pl.kernel,
        mesh=mesh,
        out_type=jax.ShapeDtypeStruct((A, B), jnp.float32),
        scratch_types=[
            pltpu.VMEM((ach, DEG, bch), jnp.int32),
            pltpu.VMEM((ach, bch), jnp.float32),
        ],
    )(_sc_mask_body)(edges_t)


def _tc_body(mask_ref, atoms_ref, bonds_ref, wa_ref, wb_ref, bias_ref, out_ref):
    maskf = mask_ref[...]
    pb = jnp.sum(bonds_ref[...] * maskf[:, None, None, :], axis=(0, 1))
    maskt = maskf.T
    pa = jnp.sum(atoms_ref[...] * maskt[:, :, None], axis=1)
    cnt = jnp.sum(maskt, axis=1)
    out = jnp.dot(pa, wa_ref[...], preferred_element_type=jnp.float32)
    out += jax.lax.dot_general(pb, wb_ref[...], (((0,), (0,)), ((), ())),
                               preferred_element_type=jnp.float32)
    out += cnt[:, None] * bias_ref[...]
    out_ref[...] = out


def kernel(atoms, bonds, edges, W, b):
    B, A, AF = atoms.shape
    DEG, BF = bonds.shape[2], bonds.shape[3]
    FP = W.shape[1]
    BB = 128

    bonds_t = jnp.transpose(bonds, (1, 2, 3, 0))
    edges_t = jnp.transpose(edges, (1, 2, 0))
    wa = W[:AF]
    wb = W[AF:]
    bias = b.reshape(1, FP)

    mask = _sc_mask(edges_t)

    return pl.pallas_call(
        _tc_body,
        grid=(B // BB,),
        in_specs=[
            pl.BlockSpec((A, BB), lambda i: (0, i)),
            pl.BlockSpec((BB, A, AF), lambda i: (i, 0, 0)),
            pl.BlockSpec((A, DEG, BF, BB), lambda i: (0, 0, 0, i)),
            pl.BlockSpec((AF, FP), lambda i: (0, 0)),
            pl.BlockSpec((BF, FP), lambda i: (0, 0)),
            pl.BlockSpec((1, FP), lambda i: (0, 0)),
        ],
        out_specs=pl.BlockSpec((BB, FP), lambda i: (i, 0)),
        out_shape=jax.ShapeDtypeStruct((B, FP), jnp.float32),
    )(mask, atoms, bonds_t, wa, wb, bias)


# speedup vs baseline: 1.4671x; 1.4671x over previous
import jax
import jax.numpy as jnp
from jax.experimental import pallas as pl
from jax.experimental.pallas import tpu as pltpu


def _body(edges_ref, atoms_ref, bonds_ref, wa_ref, wb_ref, bias_ref, out_ref,
          pa_ref, pb_ref):
    j = pl.program_id(1)
    nj = pl.num_programs(1)
    edges = edges_ref[...]
    mask = jnp.any(edges != -1, axis=1)
    maskf = mask.astype(jnp.float32)
    pb = jnp.sum(bonds_ref[...] * maskf[:, None, None, :], axis=(0, 1))
    maskt = maskf.T
    pa = jnp.sum(atoms_ref[...] * maskt[:, :, None], axis=1)
    cnt = jnp.sum(maskt, axis=1)
    out = jnp.dot(pa, wa_ref[...], preferred_element_type=jnp.float32)
    out += jax.lax.dot_general(pb, wb_ref[...], (((0,), (0,)), ((), ())),
                               preferred_element_type=jnp.float32)
    out += cnt[:, None] * bias_ref[...]

    @pl.when(j == 0)
    def _():
        pa_ref[...] = out

    @pl.when(j > 0)
    def _():
        pa_ref[...] += out

    @pl.when(j == nj - 1)
    def _():
        out_ref[...] = pa_ref[...]
    del pb_ref


def kernel(atoms, bonds, edges, W, b):
    B, A, AF = atoms.shape
    DEG, BF = bonds.shape[2], bonds.shape[3]
    FP = W.shape[1]
    BB = 128
    AC = A // 2

    bonds_t = jnp.transpose(bonds, (1, 2, 3, 0))
    edges_t = jnp.transpose(edges, (1, 2, 0))
    wa = W[:AF]
    wb = W[AF:]
    bias = b.reshape(1, FP)

    return pl.pallas_call(
        _body,
        grid=(B // BB, A // AC),
        in_specs=[
            pl.BlockSpec((AC, DEG, BB), lambda i, j: (j, 0, i)),
            pl.BlockSpec((BB, AC, AF), lambda i, j: (i, j, 0)),
            pl.BlockSpec((AC, DEG, BF, BB), lambda i, j: (j, 0, 0, i)),
            pl.BlockSpec((AF, FP), lambda i, j: (0, 0)),
            pl.BlockSpec((BF, FP), lambda i, j: (0, 0)),
            pl.BlockSpec((1, FP), lambda i, j: (0, 0)),
        ],
        out_specs=pl.BlockSpec((BB, FP), lambda i, j: (i, 0)),
        out_shape=jax.ShapeDtypeStruct((B, FP), jnp.float32),
        scratch_shapes=[
            pltpu.VMEM((BB, FP), jnp.float32),
            pltpu.VMEM((BF, BB), jnp.float32),
        ],
    )(edges_t, atoms, bonds_t, wa, wb, bias)
